# baseline (speedup 1.0000x reference)
import dataclasses

import jax
import jax.numpy as jnp
from jax import lax
from jax.experimental import pallas as pl
from jax.experimental.pallas import tpu as pltpu
from jax.experimental.pallas import tpu_sc as plsc

B = 512
D = 128
TRASH = 512

N_ATOMS = 10000
N_BONDS = 160000

GB = 16
GA = 16
N_ATOMS_PAD = 10240
NG_B = N_BONDS // GB
NG_A = N_ATOMS_PAD // GA

NW = 32

CH_B = NG_B // 16
CH_A = NG_A // 16
K1B_FLOOR, K1B_EXTRA = CH_B // NW, CH_B % NW
K1A_FLOOR, K1A_EXTRA = CH_A // NW, CH_A % NW

C2_B = NG_B // 80
C2_A = NG_A // 80
K2B_FLOOR, K2B_EXTRA = C2_B // NW, C2_B % NW
ACC2 = 528


def _compiler_params():
    cp = pltpu.CompilerParams()
    if "needs_layout_passes" in pltpu.CompilerParams.__dataclass_fields__:
        cp = dataclasses.replace(cp, needs_layout_passes=False)
    return cp


def _bond_gsum_body(x_ref, o_ref):
    acc = x_ref[:, 0, :]
    for k in range(2, 2 * GB, 2):
        acc = acc + x_ref[:, k, :]
    o_ref[...] = acc


def _atom_gsum_body(x_ref, o_ref):
    acc = x_ref[:, 0, :]
    for k in range(1, GA):
        acc = acc + x_ref[:, k, :]
    o_ref[...] = acc


def _group_sums(bond_feats, atom_feats):
    bonds3 = bond_feats.reshape(NG_B, 2 * GB, D)
    gsum_b = pl.pallas_call(
        _bond_gsum_body,
        grid=(50,),
        in_specs=[pl.BlockSpec((200, 2 * GB, D), lambda i: (i, 0, 0))],
        out_specs=pl.BlockSpec((200, D), lambda i: (i, 0)),
        out_shape=jax.ShapeDtypeStruct((NG_B, D), jnp.float32),
    )(bonds3)
    n_real = atom_feats.shape[0] // GA
    atoms3 = atom_feats.reshape(n_real, GA, D)
    gsum_a = pl.pallas_call(
        _atom_gsum_body,
        grid=(1,),
        in_specs=[pl.BlockSpec((n_real, GA, D), lambda i: (0, 0, 0))],
        out_specs=pl.BlockSpec((n_real, D), lambda i: (0, 0)),
        out_shape=jax.ShapeDtypeStruct((n_real, D), jnp.float32),
    )(atoms3)
    gsum_a = jnp.concatenate(
        [gsum_a, jnp.zeros((NG_A - n_real, D), jnp.float32)], axis=0
    )
    return gsum_b, gsum_a


def _lane_iota():
    return lax.broadcasted_iota(jnp.int32, (16,), 0)


def _extract(vec, k):
    sel = jnp.where(_lane_iota() == k, vec, 0)
    return lax.reduce_sum_p.bind(sel, axes=(0,))


def _sc1_body(
    bond_hbm,
    bseg_hbm,
    atom_hbm,
    aseg_hbm,
    destb_hbm,
    desta_hbm,
    outb_hbm,
    outa_hbm,
    acc_a,
    acc_b,
    cids_b,
    cids_a,
    dst_b,
    dst_a,
    brow_v,
    arow_v,
    gidx_v,
    seg16_v,
    seg8_v,
    mlist_v,
    tmp_v,
    sem_pre,
    sem_dst,
):
    cid = lax.axis_index("c")
    sid = lax.axis_index("s")
    wid = sid * 2 + cid

    nb1 = K1B_FLOOR + jnp.where(wid < K1B_EXTRA, 1, 0)
    na1 = K1A_FLOOR + jnp.where(wid < K1A_EXTRA, 1, 0)

    @pl.loop(0, nb1)
    def _(j):
        c = j * NW + wid
        pltpu.async_copy(bseg_hbm.at[pl.ds(c * 256, 256)], cids_b.at[j], sem_pre)

    @pl.loop(0, na1)
    def _(j):
        c = j * NW + wid
        pltpu.async_copy(aseg_hbm.at[pl.ds(c * 256, 256)], cids_a.at[j], sem_pre)

    @pl.loop(0, 32)
    def _(r):
        @pl.loop(0, D // 16)
        def _(cc):
            tmp_v[r, pl.ds(cc * 16, 16)] = jnp.zeros((16,), jnp.float32)

    pltpu.sync_copy(tmp_v, acc_a.at[pl.ds(sid * 32, 32)])
    pltpu.sync_copy(tmp_v, acc_b.at[pl.ds(sid * 32, 32)])

    @pl.loop(0, nb1)
    def _(j):
        pltpu.make_async_copy(bseg_hbm.at[pl.ds(0, 256)], cids_b.at[0], sem_pre).wait()

    @pl.loop(0, na1)
    def _(j):
        pltpu.make_async_copy(aseg_hbm.at[pl.ds(0, 256)], cids_a.at[0], sem_pre).wait()

    plsc.subcore_barrier()

    lane = _lane_iota()

    @pl.loop(0, nb1)
    def _(j):
        c = j * NW + wid
        jv = jnp.full((16,), j, jnp.int32)
        f = plsc.load_gather(cids_b.at[:, :], [jv, lane * GB])
        l = plsc.load_gather(cids_b.at[:, :], [jv, lane * GB + (GB - 1)])
        mixed = f != l
        dst_b[j, :] = jnp.where(mixed, TRASH, f)
        pltpu.async_copy(dst_b.at[j], destb_hbm.at[pl.ds(c * 16, 16)], sem_dst)
        cnt = lax.reduce_sum_p.bind(jnp.where(mixed, 1, 0), axes=(0,))
        plsc.store_compressed(mlist_v.at[:], lane, mask=mixed)

        @pl.loop(0, cnt)
        def _(k):
            moff = _extract(mlist_v[...], k)
            grow = (c * 16 + moff) * GB
            gidx_v[...] = 2 * grow + 2 * lane
            pltpu.sync_copy(bond_hbm.at[gidx_v], brow_v)
            seg16_v[...] = plsc.load_gather(
                cids_b.at[:, :], [jv, moff * GB + lane]
            )
            pltpu.sync_copy(brow_v, acc_b.at[seg16_v], add=True)

    @pl.loop(0, na1)
    def _(j):
        c = j * NW + wid
        jv = jnp.full((16,), j, jnp.int32)
        f = plsc.load_gather(cids_a.at[:, :], [jv, lane * GA])
        l = plsc.load_gather(cids_a.at[:, :], [jv, lane * GA + (GA - 1)])
        mixed = f != l
        dst_a[j, :] = jnp.where(mixed, TRASH, f)
        pltpu.async_copy(dst_a.at[j], desta_hbm.at[pl.ds(c * 16, 16)], sem_dst)
        cnt = lax.reduce_sum_p.bind(jnp.where(mixed, 1, 0), axes=(0,))
        plsc.store_compressed(mlist_v.at[:], lane, mask=mixed)

        @pl.loop(0, cnt)
        def _(k):
            moff = _extract(mlist_v[...], k)
            grow = (c * 16 + moff) * GA
            pltpu.sync_copy(atom_hbm.at[pl.ds(grow, GA)], arow_v)
            seg8_v[...] = plsc.load_gather(
                cids_a.at[:, :], [jv, moff * GA + lane]
            )
            pltpu.sync_copy(arow_v, acc_a.at[seg8_v], add=True)

    @pl.loop(0, nb1)
    def _(j):
        pltpu.make_async_copy(dst_b.at[0], destb_hbm.at[pl.ds(0, 16)], sem_dst).wait()

    @pl.loop(0, na1)
    def _(j):
        pltpu.make_async_copy(dst_a.at[0], desta_hbm.at[pl.ds(0, 16)], sem_dst).wait()

    plsc.subcore_barrier()

    pltpu.sync_copy(acc_a.at[pl.ds(sid * 32, 32)], tmp_v)
    pltpu.sync_copy(tmp_v, outa_hbm.at[cid, pl.ds(sid * 32, 32)])
    pltpu.sync_copy(acc_b.at[pl.ds(sid * 32, 32)], tmp_v)
    pltpu.sync_copy(tmp_v, outb_hbm.at[cid, pl.ds(sid * 32, 32)])


def _sc1(bond_feats, b_ids, atom_feats, a_ids):
    mesh = plsc.VectorSubcoreMesh(core_axis_name="c", subcore_axis_name="s")
    f32, i32 = jnp.float32, jnp.int32
    kern = pl.kernel(
        _sc1_body,
        out_type=(
            jax.ShapeDtypeStruct((NG_B,), i32),
            jax.ShapeDtypeStruct((NG_A,), i32),
            jax.ShapeDtypeStruct((2, B, D), f32),
            jax.ShapeDtypeStruct((2, B, D), f32),
        ),
        mesh=mesh,
        compiler_params=_compiler_params(),
        scratch_types=[
            pltpu.VMEM_SHARED((B, D), f32),
            pltpu.VMEM_SHARED((B, D), f32),
            pltpu.VMEM((K1B_FLOOR + 1, 256), i32),
            pltpu.VMEM((K1A_FLOOR + 1, 256), i32),
            pltpu.VMEM((K1B_FLOOR + 1, 16), i32),
            pltpu.VMEM((K1A_FLOOR + 1, 16), i32),
            pltpu.VMEM((GB, D), f32),
            pltpu.VMEM((GA, D), f32),
            pltpu.VMEM((16,), i32),
            pltpu.VMEM((16,), i32),
            pltpu.VMEM((GA,), i32),
            pltpu.VMEM((16,), i32),
            pltpu.VMEM((32, D), f32),
            pltpu.SemaphoreType.DMA,
            pltpu.SemaphoreType.DMA,
        ],
    )
    return kern(bond_feats, b_ids, atom_feats, a_ids)


def _sc2_body(
    gsb_hbm,
    gsa_hbm,
    destb_hbm,
    desta_hbm,
    outb_hbm,
    outa_hbm,
    acc_a,
    acc_b,
    rows0,
    rows1,
    didx0,
    didx1,
    tmp_v,
    gsem0,
    gsem1,
    dsem0,
    dsem1,
):
    cid = lax.axis_index("c")
    sid = lax.axis_index("s")
    wid = sid * 2 + cid

    n2b = K2B_FLOOR + jnp.where(wid < K2B_EXTRA, 1, 0)

    @pl.loop(0, 33)
    def _(r):
        @pl.loop(0, D // 16)
        def _(cc):
            tmp_v[r, pl.ds(cc * 16, 16)] = jnp.zeros((16,), jnp.float32)

    pltpu.sync_copy(tmp_v, acc_a.at[pl.ds(sid * 33, 33)])
    pltpu.sync_copy(tmp_v, acc_b.at[pl.ds(sid * 33, 33)])
    plsc.subcore_barrier()

    lane = _lane_iota()

    def bond_start(j, rbuf, gsem, dbuf, dsem):
        g0 = (j * NW + wid) * 80
        pltpu.async_copy(gsb_hbm.at[pl.ds(g0, 80)], rbuf, gsem)
        pltpu.async_copy(destb_hbm.at[pl.ds(g0, 80)], dbuf, dsem)

    def bond_step(j, rbuf, gsem, dbuf, dsem, nrbuf, ngsem, ndbuf, ndsem):
        @pl.when(j < n2b)
        def _():
            @pl.when(j + 1 < n2b)
            def _():
                bond_start(j + 1, nrbuf, ngsem, ndbuf, ndsem)

            pltpu.make_async_copy(gsb_hbm.at[pl.ds(0, 80)], rbuf, gsem).wait()
            pltpu.make_async_copy(destb_hbm.at[pl.ds(0, 80)], dbuf, dsem).wait()
            pltpu.sync_copy(rbuf, acc_b.at[dbuf], add=True)

    bond_start(0, rows0, gsem0, didx0, dsem0)

    @pl.loop(0, K2B_FLOOR + 1, step=2)
    def _(j):
        bond_step(j, rows0, gsem0, didx0, dsem0, rows1, gsem1, didx1, dsem1)
        bond_step(j + 1, rows1, gsem1, didx1, dsem1, rows0, gsem0, didx0, dsem0)

    @pl.when(wid < C2_A)
    def _():
        g0 = wid * 80
        pltpu.async_copy(gsa_hbm.at[pl.ds(g0, 80)], rows0, gsem0)
        pltpu.sync_copy(desta_hbm.at[pl.ds(g0, 80)], didx0)
        pltpu.make_async_copy(gsa_hbm.at[pl.ds(0, 80)], rows0, gsem0).wait()
        pltpu.sync_copy(rows0, acc_a.at[didx0], add=True)

    plsc.subcore_barrier()

    pltpu.sync_copy(acc_a.at[pl.ds(sid * 32, 32)], tmp_v.at[pl.ds(0, 32)])
    pltpu.sync_copy(tmp_v.at[pl.ds(0, 32)], outa_hbm.at[cid, pl.ds(sid * 32, 32)])
    pltpu.sync_copy(acc_b.at[pl.ds(sid * 32, 32)], tmp_v.at[pl.ds(0, 32)])
    pltpu.sync_copy(tmp_v.at[pl.ds(0, 32)], outb_hbm.at[cid, pl.ds(sid * 32, 32)])


def _sc2(gsum_b, gsum_a, dest_b, dest_a):
    mesh = plsc.VectorSubcoreMesh(core_axis_name="c", subcore_axis_name="s")
    f32, i32 = jnp.float32, jnp.int32
    kern = pl.kernel(
        _sc2_body,
        out_type=(
            jax.ShapeDtypeStruct((2, B, D), f32),
            jax.ShapeDtypeStruct((2, B, D), f32),
        ),
        mesh=mesh,
        compiler_params=_compiler_params(),
        scratch_types=[
            pltpu.VMEM_SHARED((ACC2, D), f32),
            pltpu.VMEM_SHARED((ACC2, D), f32),
            pltpu.VMEM((80, D), f32),
            pltpu.VMEM((80, D), f32),
            pltpu.VMEM((80,), i32),
            pltpu.VMEM((80,), i32),
            pltpu.VMEM((33, D), f32),
            pltpu.SemaphoreType.DMA,
            pltpu.SemaphoreType.DMA,
            pltpu.SemaphoreType.DMA,
            pltpu.SemaphoreType.DMA,
        ],
    )
    return kern(gsum_b, gsum_a, dest_b, dest_a)


def _combine_body(ap1, bp1, ap2, bp2, g_ref, out_ref):
    out_ref[:, 0:D] = ap1[0] + ap1[1] + ap2[0] + ap2[1]
    out_ref[:, D : 2 * D] = bp1[0] + bp1[1] + bp2[0] + bp2[1]
    out_ref[:, 2 * D : 3 * D] = g_ref[...]


def _combine(ap1, bp1, ap2, bp2, g):
    return pl.pallas_call(
        _combine_body,
        out_shape=jax.ShapeDtypeStruct((B, 3 * D), jnp.float32),
    )(ap1, bp1, ap2, bp2, g)


def kernel(atom_feats, bond_feats, global_feats, atom_segment_ids, bond_segment_ids):
    n_pad = N_ATOMS_PAD - atom_feats.shape[0]
    a_ids = jnp.concatenate(
        [
            atom_segment_ids.astype(jnp.int32),
            jnp.full((n_pad,), B - 1, jnp.int32),
        ]
    )
    b_ids = bond_segment_ids.astype(jnp.int32)
    gsum_b, gsum_a = _group_sums(bond_feats, atom_feats)
    dest_b, dest_a, bp1, ap1 = _sc1(bond_feats, b_ids, atom_feats, a_ids)
    gbp, gap = _sc2(gsum_b, gsum_a, dest_b, dest_a)
    return _combine(ap1, bp1, gap, gbp, global_feats)


# speedup vs baseline: 1.3976x; 1.3976x over previous
import dataclasses

import jax
import jax.numpy as jnp
from jax import lax
from jax.experimental import pallas as pl
from jax.experimental.pallas import tpu as pltpu
from jax.experimental.pallas import tpu_sc as plsc

B = 512
D = 128

N_ATOMS = 10000
N_BONDS = 160000

BBLK = 128
ABLK = 80
NB_BOND = N_BONDS // BBLK
NB_ATOM = N_ATOMS // ABLK
NW = 32
BOND_FLOOR = NB_BOND // NW
ATOM_FLOOR = NB_ATOM // NW
BOND_MAX = BOND_FLOOR + 1
ATOM_MAX = ATOM_FLOOR + 1


def _sc_pool_body(
    bond_hbm,
    bseg_hbm,
    atom_hbm,
    aseg_hbm,
    out_a_hbm,
    out_b_hbm,
    acc_a,
    acc_b,
    rows0,
    rows1,
    arows0,
    arows1,
    idxA,
    idxB,
    bseg_all,
    aseg_all,
    tmp_v,
    sem_pre,
    gsem0,
    gsem1,
):
    cid = lax.axis_index("c")
    sid = lax.axis_index("s")
    wid = sid * 2 + cid

    nb = BOND_FLOOR + jnp.where(wid < NB_BOND - BOND_FLOOR * NW, 1, 0)
    na = ATOM_FLOOR + jnp.where(wid < NB_ATOM - ATOM_FLOOR * NW, 1, 0)

    @pl.loop(0, nb)
    def _(j):
        row0 = (j * NW + wid) * BBLK
        pltpu.async_copy(bseg_hbm.at[pl.ds(row0, BBLK)], bseg_all.at[j], sem_pre)

    @pl.loop(0, na)
    def _(j):
        row0 = (j * NW + wid) * ABLK
        pltpu.async_copy(aseg_hbm.at[pl.ds(row0, ABLK)], aseg_all.at[j], sem_pre)

    @pl.loop(0, 32)
    def _(r):
        @pl.loop(0, D // 16)
        def _(c):
            tmp_v[r, pl.ds(c * 16, 16)] = jnp.zeros((16,), jnp.float32)

    pltpu.sync_copy(tmp_v, acc_a.at[pl.ds(sid * 32, 32)])
    pltpu.sync_copy(tmp_v, acc_b.at[pl.ds(sid * 32, 32)])

    @pl.loop(0, nb)
    def _(j):
        pltpu.make_async_copy(bseg_hbm.at[pl.ds(0, BBLK)], bseg_all.at[0], sem_pre).wait()

    lane = lax.broadcasted_iota(jnp.int32, (16,), 0)
    for c in range(BBLK // 16):
        idxA[pl.ds(c * 16, 16)] = 2 * (wid * BBLK + c * 16) + 2 * lane
        idxB[pl.ds(c * 16, 16)] = 2 * ((NW + wid) * BBLK + c * 16) + 2 * lane

    @pl.loop(0, na)
    def _(j):
        pltpu.make_async_copy(aseg_hbm.at[pl.ds(0, ABLK)], aseg_all.at[0], sem_pre).wait()

    plsc.subcore_barrier()

    def bond_step(j, buf, sem, idxbuf, nxt_buf, nxt_sem, nxt_idx):
        @pl.when(j < nb)
        def _():
            @pl.when(j + 1 < nb)
            def _():
                pltpu.async_copy(bond_hbm.at[nxt_idx], nxt_buf, nxt_sem)

            pltpu.make_async_copy(bond_hbm.at[idxbuf], buf, sem).wait()
            for c in range(BBLK // 16):
                idxbuf[pl.ds(c * 16, 16)] = idxbuf[pl.ds(c * 16, 16)] + 4 * NW * BBLK

            pltpu.sync_copy(buf, acc_b.at[bseg_all.at[j]], add=True)

    pltpu.async_copy(bond_hbm.at[idxA], rows0, gsem0)

    @pl.loop(0, BOND_MAX, step=2)
    def _(j):
        bond_step(j, rows0, gsem0, idxA, rows1, gsem1, idxB)
        bond_step(j + 1, rows1, gsem1, idxB, rows0, gsem0, idxA)

    def atom_gather(j, buf, sem):
        row0 = (j * NW + wid) * ABLK
        pltpu.async_copy(atom_hbm.at[pl.ds(row0, ABLK)], buf, sem)

    def atom_step(j, buf, sem, nxt_buf, nxt_sem):
        @pl.when(j < na)
        def _():
            @pl.when(j + 1 < na)
            def _():
                atom_gather(j + 1, nxt_buf, nxt_sem)

            pltpu.make_async_copy(atom_hbm.at[pl.ds(0, ABLK)], buf, sem).wait()
            pltpu.sync_copy(buf, acc_a.at[aseg_all.at[j]], add=True)

    atom_gather(0, arows0, gsem0)

    @pl.loop(0, ATOM_MAX, step=2)
    def _(j):
        atom_step(j, arows0, gsem0, arows1, gsem1)
        atom_step(j + 1, arows1, gsem1, arows0, gsem0)

    plsc.subcore_barrier()

    pltpu.sync_copy(acc_a.at[pl.ds(sid * 32, 32)], tmp_v)
    pltpu.sync_copy(tmp_v, out_a_hbm.at[cid, pl.ds(sid * 32, 32)])
    pltpu.sync_copy(acc_b.at[pl.ds(sid * 32, 32)], tmp_v)
    pltpu.sync_copy(tmp_v, out_b_hbm.at[cid, pl.ds(sid * 32, 32)])


def _sc_pool(bond_feats, b_ids, atom_feats, a_ids):
    mesh = plsc.VectorSubcoreMesh(core_axis_name="c", subcore_axis_name="s")
    f32 = jnp.float32
    i32 = jnp.int32
    cp = pltpu.CompilerParams()
    if "needs_layout_passes" in pltpu.CompilerParams.__dataclass_fields__:
        cp = dataclasses.replace(cp, needs_layout_passes=False)
    kern = pl.kernel(
        _sc_pool_body,
        compiler_params=cp,
        out_type=(
            jax.ShapeDtypeStruct((2, B, D), f32),
            jax.ShapeDtypeStruct((2, B, D), f32),
        ),
        mesh=mesh,
        scratch_types=[
            pltpu.VMEM_SHARED((B, D), f32),
            pltpu.VMEM_SHARED((B, D), f32),
            pltpu.VMEM((BBLK, D), f32),
            pltpu.VMEM((BBLK, D), f32),
            pltpu.VMEM((ABLK, D), f32),
            pltpu.VMEM((ABLK, D), f32),
            pltpu.VMEM((BBLK,), i32),
            pltpu.VMEM((BBLK,), i32),
            pltpu.VMEM((BOND_MAX, BBLK), i32),
            pltpu.VMEM((ATOM_MAX, ABLK), i32),
            pltpu.VMEM((32, D), f32),
            pltpu.SemaphoreType.DMA,
            pltpu.SemaphoreType.DMA,
            pltpu.SemaphoreType.DMA,
        ],
    )
    return kern(bond_feats, b_ids, atom_feats, a_ids)


def _combine_body(pa_ref, pb_ref, g_ref, out_ref):
    out_ref[:, 0:D] = pa_ref[0] + pa_ref[1]
    out_ref[:, D : 2 * D] = pb_ref[0] + pb_ref[1]
    out_ref[:, 2 * D : 3 * D] = g_ref[...]


def _combine(pa, pb, g):
    return pl.pallas_call(
        _combine_body,
        out_shape=jax.ShapeDtypeStruct((B, 3 * D), jnp.float32),
    )(pa, pb, g)


def kernel(atom_feats, bond_feats, global_feats, atom_segment_ids, bond_segment_ids):
    a_ids = atom_segment_ids.astype(jnp.int32)
    b_ids = bond_segment_ids.astype(jnp.int32)
    pa, pb = _sc_pool(bond_feats, b_ids, atom_feats, a_ids)
    return _combine(pa, pb, global_feats)


# speedup vs baseline: 1.5204x; 1.0878x over previous
import dataclasses

import jax
import jax.numpy as jnp
from jax import lax
from jax.experimental import pallas as pl
from jax.experimental.pallas import tpu as pltpu
from jax.experimental.pallas import tpu_sc as plsc

B = 512
D = 128

N_ATOMS = 10000
N_BONDS = 160000

BBLK = 128
ABLK = 80
NB_BOND = N_BONDS // BBLK
NB_ATOM = N_ATOMS // ABLK
NW = 32
BOND_FLOOR = NB_BOND // NW
ATOM_FLOOR = NB_ATOM // NW
BOND_MAX = BOND_FLOOR + 1
ATOM_MAX = ATOM_FLOOR + 1


def _sc_pool_body(
    bond_hbm,
    bseg_hbm,
    atom_hbm,
    aseg_hbm,
    out_a_hbm,
    out_b_hbm,
    acc_a,
    acc_b,
    rows0,
    rows1,
    rows2,
    arows0,
    arows1,
    idxA,
    idxB,
    idxC,
    bseg_all,
    aseg_all,
    tmp_v,
    sem_pre,
    gsem0,
    gsem1,
    gsem2,
    ssem0,
    ssem1,
    ssem2,
):
    cid = lax.axis_index("c")
    sid = lax.axis_index("s")
    wid = sid * 2 + cid

    nb = BOND_FLOOR + jnp.where(wid < NB_BOND - BOND_FLOOR * NW, 1, 0)
    na = ATOM_FLOOR + jnp.where(wid < NB_ATOM - ATOM_FLOOR * NW, 1, 0)

    @pl.loop(0, nb)
    def _(j):
        row0 = (j * NW + wid) * BBLK
        pltpu.async_copy(bseg_hbm.at[pl.ds(row0, BBLK)], bseg_all.at[j], sem_pre)

    @pl.loop(0, na)
    def _(j):
        row0 = (j * NW + wid) * ABLK
        pltpu.async_copy(aseg_hbm.at[pl.ds(row0, ABLK)], aseg_all.at[j], sem_pre)

    @pl.loop(0, 32)
    def _(r):
        @pl.loop(0, D // 16)
        def _(c):
            tmp_v[r, pl.ds(c * 16, 16)] = jnp.zeros((16,), jnp.float32)

    pltpu.sync_copy(tmp_v, acc_a.at[pl.ds(sid * 32, 32)])
    pltpu.sync_copy(tmp_v, acc_b.at[pl.ds(sid * 32, 32)])

    @pl.loop(0, nb)
    def _(j):
        pltpu.make_async_copy(bseg_hbm.at[pl.ds(0, BBLK)], bseg_all.at[0], sem_pre).wait()

    lane = lax.broadcasted_iota(jnp.int32, (16,), 0)
    for c in range(BBLK // 16):
        idxA[pl.ds(c * 16, 16)] = 2 * (wid * BBLK + c * 16) + 2 * lane
        idxB[pl.ds(c * 16, 16)] = 2 * ((NW + wid) * BBLK + c * 16) + 2 * lane
        idxC[pl.ds(c * 16, 16)] = 2 * ((2 * NW + wid) * BBLK + c * 16) + 2 * lane

    @pl.loop(0, na)
    def _(j):
        pltpu.make_async_copy(aseg_hbm.at[pl.ds(0, ABLK)], aseg_all.at[0], sem_pre).wait()

    plsc.subcore_barrier()

    rowsv = [rows0, rows1, rows2]
    idxv = [idxA, idxB, idxC]
    gsemv = [gsem0, gsem1, gsem2]
    ssemv = [ssem0, ssem1, ssem2]

    def bond_step(j, o):
        jj = j + o
        k = o % 3
        k2 = (o + 2) % 3

        @pl.when(jj < nb)
        def _():
            pltpu.make_async_copy(bond_hbm.at[idxv[k]], rowsv[k], gsemv[k]).wait()
            for c in range(BBLK // 16):
                idxv[k][pl.ds(c * 16, 16)] = (
                    idxv[k][pl.ds(c * 16, 16)] + 6 * NW * BBLK
                )

            @pl.when(jj + 2 < nb)
            def _():
                @pl.when(jj >= 1)
                def _():
                    pltpu.make_async_copy(
                        rowsv[k2], acc_b.at[bseg_all.at[0]], ssemv[k2]
                    ).wait()

                pltpu.async_copy(bond_hbm.at[idxv[k2]], rowsv[k2], gsemv[k2])

            pltpu.async_copy(rowsv[k], acc_b.at[bseg_all.at[jj]], ssemv[k], add=True)

    pltpu.async_copy(bond_hbm.at[idxA], rows0, gsem0)
    pltpu.async_copy(bond_hbm.at[idxB], rows1, gsem1)

    @pl.loop(0, BOND_MAX + 2, step=3)
    def _(j):
        bond_step(j, 0)
        bond_step(j, 1)
        bond_step(j, 2)

    for k in range(3):
        pltpu.make_async_copy(rowsv[k], acc_b.at[bseg_all.at[0]], ssemv[k]).wait()

    def atom_gather(j, buf, sem):
        row0 = (j * NW + wid) * ABLK
        pltpu.async_copy(atom_hbm.at[pl.ds(row0, ABLK)], buf, sem)

    def atom_step(j, buf, sem, nxt_buf, nxt_sem):
        @pl.when(j < na)
        def _():
            @pl.when(j + 1 < na)
            def _():
                atom_gather(j + 1, nxt_buf, nxt_sem)

            pltpu.make_async_copy(atom_hbm.at[pl.ds(0, ABLK)], buf, sem).wait()
            pltpu.sync_copy(buf, acc_a.at[aseg_all.at[j]], add=True)

    atom_gather(0, arows0, gsem0)

    @pl.loop(0, ATOM_MAX, step=2)
    def _(j):
        atom_step(j, arows0, gsem0, arows1, gsem1)
        atom_step(j + 1, arows1, gsem1, arows0, gsem0)

    plsc.subcore_barrier()

    pltpu.sync_copy(acc_a.at[pl.ds(sid * 32, 32)], tmp_v)
    pltpu.sync_copy(tmp_v, out_a_hbm.at[cid, pl.ds(sid * 32, 32)])
    pltpu.sync_copy(acc_b.at[pl.ds(sid * 32, 32)], tmp_v)
    pltpu.sync_copy(tmp_v, out_b_hbm.at[cid, pl.ds(sid * 32, 32)])


def _sc_pool(bond_feats, b_ids, atom_feats, a_ids):
    mesh = plsc.VectorSubcoreMesh(core_axis_name="c", subcore_axis_name="s")
    f32 = jnp.float32
    i32 = jnp.int32
    cp = pltpu.CompilerParams()
    if "needs_layout_passes" in pltpu.CompilerParams.__dataclass_fields__:
        cp = dataclasses.replace(cp, needs_layout_passes=False)
    kern = pl.kernel(
        _sc_pool_body,
        compiler_params=cp,
        out_type=(
            jax.ShapeDtypeStruct((2, B, D), f32),
            jax.ShapeDtypeStruct((2, B, D), f32),
        ),
        mesh=mesh,
        scratch_types=[
            pltpu.VMEM_SHARED((B, D), f32),
            pltpu.VMEM_SHARED((B, D), f32),
            pltpu.VMEM((BBLK, D), f32),
            pltpu.VMEM((BBLK, D), f32),
            pltpu.VMEM((BBLK, D), f32),
            pltpu.VMEM((ABLK, D), f32),
            pltpu.VMEM((ABLK, D), f32),
            pltpu.VMEM((BBLK,), i32),
            pltpu.VMEM((BBLK,), i32),
            pltpu.VMEM((BBLK,), i32),
            pltpu.VMEM((BOND_MAX, BBLK), i32),
            pltpu.VMEM((ATOM_MAX, ABLK), i32),
            pltpu.VMEM((32, D), f32),
            pltpu.SemaphoreType.DMA,
            pltpu.SemaphoreType.DMA,
            pltpu.SemaphoreType.DMA,
            pltpu.SemaphoreType.DMA,
            pltpu.SemaphoreType.DMA,
            pltpu.SemaphoreType.DMA,
            pltpu.SemaphoreType.DMA,
        ],
    )
    return kern(bond_feats, b_ids, atom_feats, a_ids)


def _combine_body(pa_ref, pb_ref, g_ref, out_ref):
    out_ref[:, 0:D] = pa_ref[0] + pa_ref[1]
    out_ref[:, D : 2 * D] = pb_ref[0] + pb_ref[1]
    out_ref[:, 2 * D : 3 * D] = g_ref[...]


def _combine(pa, pb, g):
    return pl.pallas_call(
        _combine_body,
        out_shape=jax.ShapeDtypeStruct((B, 3 * D), jnp.float32),
    )(pa, pb, g)


def kernel(atom_feats, bond_feats, global_feats, atom_segment_ids, bond_segment_ids):
    a_ids = atom_segment_ids.astype(jnp.int32)
    b_ids = bond_segment_ids.astype(jnp.int32)
    pa, pb = _sc_pool(bond_feats, b_ids, atom_feats, a_ids)
    return _combine(pa, pb, global_feats)
